# baseline (speedup 1.0000x reference)
import functools
import jax
import jax.numpy as jnp
from jax.experimental import pallas as pl
from jax.experimental.pallas import tpu as pltpu

_IDX_BITS = 9
_KEY_MASK = -(1 << _IDX_BITS)
_BIAS = 1 << 23


def _fp_body(xt_ref, xs_ref, ft_ref, fs_ref, w1a_ref, w1b_ref, w2_ref,
             out_ref):
    n_t = xt_ref.shape[1]
    n_s = xs_ref.shape[2]

    xt = xt_ref[0]
    xs = xs_ref[0]
    tn2 = jnp.sum(xt * xt, axis=1, keepdims=True)
    sn2 = jnp.sum(xs * xs, axis=0, keepdims=True)
    ts = jnp.dot(xt, xs, preferred_element_type=jnp.float32,
                 precision=jax.lax.Precision.HIGHEST)
    d2 = jnp.maximum(tn2 + sn2 - 2.0 * ts, 0.0)

    s_iota = jax.lax.broadcasted_iota(jnp.int32, (n_t, n_s), 1)
    keyi = ((jax.lax.bitcast_convert_type(d2, jnp.int32)
             + (1 << (_IDX_BITS - 1))) & _KEY_MASK) | s_iota
    keyf = jax.lax.bitcast_convert_type(keyi + _BIAS, jnp.float32)

    masked = keyf
    mks = []
    for r in range(3):
        mk = jnp.min(masked, axis=1, keepdims=True)
        mks.append(mk)
        if r < 2:
            masked = jnp.where(masked == mk, jnp.inf, masked)

    rs = []
    for mk in mks:
        bits = jax.lax.bitcast_convert_type(mk, jnp.int32) - _BIAS
        d2k = jax.lax.bitcast_convert_type(bits & _KEY_MASK, jnp.float32)
        rs.append(jax.lax.rsqrt(jnp.maximum(d2k, 1e-20)))
    norm = rs[0] + rs[1] + rs[2]
    inv = 1.0 / (norm * (1.0 + 1e-6))
    cs = [r * inv for r in rs]

    zero = jnp.zeros((), jnp.float32)
    coeff = jnp.where(
        keyf == mks[0], cs[0],
        jnp.where(keyf == mks[1], cs[1],
                  jnp.where(keyf == mks[2], cs[2], zero)))

    g = jnp.dot(fs_ref[0], w1a_ref[...], preferred_element_type=jnp.float32)
    h = jnp.dot(coeff, g, preferred_element_type=jnp.float32)
    h = h + jnp.dot(ft_ref[0], w1b_ref[...],
                    preferred_element_type=jnp.float32)
    h = jnp.maximum(h, 0.0)
    out = jnp.dot(h, w2_ref[...], preferred_element_type=jnp.float32)
    out_ref[0] = jnp.maximum(out, 0.0)


@jax.jit
def kernel(xyz_target, xyz_source, feats_target, feats_source, W1, W2):
    bs, n_t, _ = xyz_target.shape
    n_s = xyz_source.shape[1]
    c_t = feats_target.shape[2]
    c_s = feats_source.shape[2]

    xt = jnp.concatenate(
        [xyz_target, jnp.zeros((bs, n_t, 5), jnp.float32)], axis=2)
    xs = jnp.concatenate(
        [jnp.transpose(xyz_source, (0, 2, 1)),
         jnp.zeros((bs, 5, n_s), jnp.float32)], axis=1)
    W1a = W1[:c_s]
    W1b = W1[c_s:]

    grid = (bs,)
    out = pl.pallas_call(
        _fp_body,
        grid=grid,
        in_specs=[
            pl.BlockSpec((1, n_t, 8), lambda b: (b, 0, 0)),
            pl.BlockSpec((1, 8, n_s), lambda b: (b, 0, 0)),
            pl.BlockSpec((1, n_t, c_t), lambda b: (b, 0, 0)),
            pl.BlockSpec((1, n_s, c_s), lambda b: (b, 0, 0)),
            pl.BlockSpec((c_s, W1.shape[1]), lambda b: (0, 0)),
            pl.BlockSpec((c_t, W1.shape[1]), lambda b: (0, 0)),
            pl.BlockSpec(W2.shape, lambda b: (0, 0)),
        ],
        out_specs=pl.BlockSpec((1, n_t, W2.shape[1]), lambda b: (b, 0, 0)),
        out_shape=jax.ShapeDtypeStruct((bs, n_t, W2.shape[1]), jnp.float32),
    )(xt, xs, feats_target, feats_source, W1a, W1b, W2)
    return out


# speedup vs baseline: 1.1698x; 1.1698x over previous
import functools
import jax
import jax.numpy as jnp
from jax.experimental import pallas as pl
from jax.experimental.pallas import tpu as pltpu

_IDX_BITS = 9
_KEY_MASK = -(1 << _IDX_BITS)
_BIAS = 1 << 23


def _fp_body(xt_ref, xs_ref, th_ref, tl_ref, sh_ref, sl_ref,
             ft_ref, fs_ref, w1a_ref, w1b_ref, w2_ref, out_ref):
    n_t = xt_ref.shape[1]
    n_s = xs_ref.shape[2]

    xt = xt_ref[0]
    xs = xs_ref[0]
    tn2 = jnp.sum(xt * xt, axis=1, keepdims=True)
    sn2 = jnp.sum(xs * xs, axis=0, keepdims=True)
    ts = jnp.dot(th_ref[0], sh_ref[0], preferred_element_type=jnp.float32)
    ts = ts + (jnp.dot(th_ref[0], sl_ref[0],
                       preferred_element_type=jnp.float32)
               + jnp.dot(tl_ref[0], sh_ref[0],
                         preferred_element_type=jnp.float32))
    d2 = jnp.maximum(tn2 + sn2 - 2.0 * ts, 0.0)

    s_iota = jax.lax.broadcasted_iota(jnp.int32, (n_t, n_s), 1)
    keyi = ((jax.lax.bitcast_convert_type(d2, jnp.int32)
             + (1 << (_IDX_BITS - 1))) & _KEY_MASK) | s_iota
    keyf = jax.lax.bitcast_convert_type(keyi + _BIAS, jnp.float32)

    masked = keyf
    mks = []
    for r in range(3):
        mk = jnp.min(masked, axis=1, keepdims=True)
        mks.append(mk)
        if r < 2:
            masked = jnp.where(masked == mk, jnp.inf, masked)

    rs = []
    for mk in mks:
        bits = jax.lax.bitcast_convert_type(mk, jnp.int32) - _BIAS
        d2k = jax.lax.bitcast_convert_type(bits & _KEY_MASK, jnp.float32)
        rs.append(jax.lax.rsqrt(jnp.maximum(d2k, 1e-20)))
    norm = rs[0] + rs[1] + rs[2]
    inv = 1.0 / (norm * (1.0 + 1e-6))
    cs = [r * inv for r in rs]

    zero = jnp.zeros((), jnp.float32)
    coeff = jnp.where(
        keyf == mks[0], cs[0],
        jnp.where(keyf == mks[1], cs[1],
                  jnp.where(keyf == mks[2], cs[2], zero)))

    g = jnp.dot(fs_ref[0], w1a_ref[...], preferred_element_type=jnp.float32)
    h = jnp.dot(coeff, g, preferred_element_type=jnp.float32)
    h = h + jnp.dot(ft_ref[0], w1b_ref[...],
                    preferred_element_type=jnp.float32)
    h = jnp.maximum(h, 0.0)
    out = jnp.dot(h, w2_ref[...], preferred_element_type=jnp.float32)
    out_ref[0] = jnp.maximum(out, 0.0)


@jax.jit
def kernel(xyz_target, xyz_source, feats_target, feats_source, W1, W2):
    bs, n_t, _ = xyz_target.shape
    n_s = xyz_source.shape[1]
    c_t = feats_target.shape[2]
    c_s = feats_source.shape[2]

    xt = jnp.concatenate(
        [xyz_target, jnp.zeros((bs, n_t, 5), jnp.float32)], axis=2)
    xs = jnp.concatenate(
        [jnp.transpose(xyz_source, (0, 2, 1)),
         jnp.zeros((bs, 5, n_s), jnp.float32)], axis=1)
    th = xt.astype(jnp.bfloat16)
    tl = (xt - th.astype(jnp.float32)).astype(jnp.bfloat16)
    sh = xs.astype(jnp.bfloat16)
    sl = (xs - sh.astype(jnp.float32)).astype(jnp.bfloat16)
    W1a = W1[:c_s]
    W1b = W1[c_s:]

    grid = (bs,)
    out = pl.pallas_call(
        _fp_body,
        grid=grid,
        in_specs=[
            pl.BlockSpec((1, n_t, 8), lambda b: (b, 0, 0)),
            pl.BlockSpec((1, 8, n_s), lambda b: (b, 0, 0)),
            pl.BlockSpec((1, n_t, 8), lambda b: (b, 0, 0)),
            pl.BlockSpec((1, n_t, 8), lambda b: (b, 0, 0)),
            pl.BlockSpec((1, 8, n_s), lambda b: (b, 0, 0)),
            pl.BlockSpec((1, 8, n_s), lambda b: (b, 0, 0)),
            pl.BlockSpec((1, n_t, c_t), lambda b: (b, 0, 0)),
            pl.BlockSpec((1, n_s, c_s), lambda b: (b, 0, 0)),
            pl.BlockSpec((c_s, W1.shape[1]), lambda b: (0, 0)),
            pl.BlockSpec((c_t, W1.shape[1]), lambda b: (0, 0)),
            pl.BlockSpec(W2.shape, lambda b: (0, 0)),
        ],
        out_specs=pl.BlockSpec((1, n_t, W2.shape[1]), lambda b: (b, 0, 0)),
        out_shape=jax.ShapeDtypeStruct((bs, n_t, W2.shape[1]), jnp.float32),
    )(xt, xs, th, tl, sh, sl, feats_target, feats_source, W1a, W1b, W2)
    return out


# speedup vs baseline: 1.4160x; 1.2105x over previous
import functools
import jax
import jax.numpy as jnp
from jax.experimental import pallas as pl
from jax.experimental.pallas import tpu as pltpu

_IDX_BITS = 9
_KEY_MASK = -(1 << _IDX_BITS)
_BIAS = 1 << 23


def _fp_body(xt_ref, xs_ref, ft_ref, fs_ref, w1a_ref, w1b_ref, w2_ref,
             out_ref):
    n_t = xt_ref.shape[1]
    n_s = xs_ref.shape[2]

    d2 = jnp.zeros((n_t, n_s), jnp.float32)
    for d in range(3):
        tcol = xt_ref[0, :, d:d + 1]
        srow = xs_ref[0, d:d + 1, :]
        diff = tcol - srow
        d2 = d2 + diff * diff

    s_iota = jax.lax.broadcasted_iota(jnp.int32, (n_t, n_s), 1)
    keyi = ((jax.lax.bitcast_convert_type(d2, jnp.int32)
             + (1 << (_IDX_BITS - 1))) & _KEY_MASK) | s_iota
    keyf = jax.lax.bitcast_convert_type(keyi + _BIAS, jnp.float32)

    masked = keyf
    mks = []
    for r in range(3):
        mk = jnp.min(masked, axis=1, keepdims=True)
        mks.append(mk)
        if r < 2:
            masked = jnp.where(masked == mk, jnp.inf, masked)

    rs = []
    for mk in mks:
        bits = jax.lax.bitcast_convert_type(mk, jnp.int32) - _BIAS
        d2k = jax.lax.bitcast_convert_type(bits & _KEY_MASK, jnp.float32)
        rs.append(jax.lax.rsqrt(jnp.maximum(d2k, 1e-20)))
    norm = rs[0] + rs[1] + rs[2]
    inv = 1.0 / (norm * (1.0 + 1e-6))
    cs = [r * inv for r in rs]

    zero = jnp.zeros((), jnp.float32)
    coeff = jnp.where(
        keyf == mks[0], cs[0],
        jnp.where(keyf == mks[1], cs[1],
                  jnp.where(keyf == mks[2], cs[2], zero)))

    g = jnp.dot(fs_ref[0], w1a_ref[...], preferred_element_type=jnp.float32)
    h = jnp.dot(coeff, g, preferred_element_type=jnp.float32)
    h = h + jnp.dot(ft_ref[0], w1b_ref[...],
                    preferred_element_type=jnp.float32)
    h = jnp.maximum(h, 0.0)
    out = jnp.dot(h, w2_ref[...], preferred_element_type=jnp.float32)
    out_ref[0] = jnp.maximum(out, 0.0)


@jax.jit
def kernel(xyz_target, xyz_source, feats_target, feats_source, W1, W2):
    bs, n_t, _ = xyz_target.shape
    n_s = xyz_source.shape[1]
    c_t = feats_target.shape[2]
    c_s = feats_source.shape[2]

    xs = jnp.transpose(xyz_source, (0, 2, 1))
    W1a = W1[:c_s]
    W1b = W1[c_s:]

    grid = (bs,)
    out = pl.pallas_call(
        _fp_body,
        grid=grid,
        in_specs=[
            pl.BlockSpec((1, n_t, 3), lambda b: (b, 0, 0)),
            pl.BlockSpec((1, 3, n_s), lambda b: (b, 0, 0)),
            pl.BlockSpec((1, n_t, c_t), lambda b: (b, 0, 0)),
            pl.BlockSpec((1, n_s, c_s), lambda b: (b, 0, 0)),
            pl.BlockSpec((c_s, W1.shape[1]), lambda b: (0, 0)),
            pl.BlockSpec((c_t, W1.shape[1]), lambda b: (0, 0)),
            pl.BlockSpec(W2.shape, lambda b: (0, 0)),
        ],
        out_specs=pl.BlockSpec((1, n_t, W2.shape[1]), lambda b: (b, 0, 0)),
        out_shape=jax.ShapeDtypeStruct((bs, n_t, W2.shape[1]), jnp.float32),
    )(xyz_target, xs, feats_target, feats_source, W1a, W1b, W2)
    return out
